# Initial kernel scaffold; baseline (speedup 1.0000x reference)
import jax
import jax.numpy as jnp
from jax.experimental import pallas as pl


def kernel(x, edge_index, W1, b1, W2, b2):
    raise NotImplementedError("write your pallas kernel here")


# speedup vs baseline: 7.3146x; 7.3146x over previous
import functools

import jax
import jax.numpy as jnp
from jax import lax
from jax.experimental import pallas as pl
from jax.experimental.pallas import tpu as pltpu
from jax.experimental.pallas import tpu_sc as plsc

N = 10000
NPAD = 10016
STRIPE = NPAD // 16
E = 160000
NTILES = 32
NCHUNK = 40
CHUNK = 128
EPAD = NTILES * NCHUNK * CHUNK
IN_DIM = 128
HID = 128
EMB = 64
BM = 1000
BN = 1280
NPAD2 = 10240
DEGW = 16


def _fill_rows(buf, nrows, d, value):
    vec = jnp.full((16,), value, jnp.float32)

    def body(r, carry):
        for k in range(d // 16):
            buf[r, pl.ds(k * 16, 16)] = vec
        return carry

    lax.fori_loop(0, nrows, body, 0)


def _zero_stripe(acc, zbuf, base):
    for off in (0, 128, 256, 384):
        pltpu.sync_copy(zbuf, acc.at[pl.ds(base + off, CHUNK)])
    rem = STRIPE - 512
    pltpu.sync_copy(zbuf.at[pl.ds(0, rem)], acc.at[pl.ds(base + 512, rem)])


def _make_deg():
    mesh = plsc.VectorSubcoreMesh(core_axis_name="c", subcore_axis_name="s")

    @functools.partial(
        pl.kernel,
        mesh=mesh,
        out_type=jax.ShapeDtypeStruct((2, 16, STRIPE, DEGW), jnp.float32),
        scratch_types=[
            pltpu.VMEM((NCHUNK, CHUNK), jnp.int32),
            pltpu.VMEM((CHUNK, DEGW), jnp.float32),
            pltpu.VMEM((CHUNK, DEGW), jnp.float32),
            pltpu.VMEM_SHARED((NPAD, DEGW), jnp.float32),
        ],
    )
    def deg_kernel(dst_hbm, out_hbm, dst_v, ones_v, zero_v, acc):
        c = lax.axis_index("c")
        s = lax.axis_index("s")
        wid = c * 16 + s
        pltpu.sync_copy(dst_hbm.at[wid], dst_v)
        _fill_rows(ones_v, CHUNK, DEGW, 1.0)
        _fill_rows(zero_v, CHUNK, DEGW, 0.0)
        _zero_stripe(acc, zero_v, s * STRIPE)
        plsc.subcore_barrier()
        for j in range(NCHUNK):
            pltpu.sync_copy(ones_v, acc.at[dst_v.at[j]], add=True)
        plsc.subcore_barrier()
        pltpu.sync_copy(acc.at[pl.ds(s * STRIPE, STRIPE)], out_hbm.at[c, s])

    return deg_kernel


def _make_agg(d):
    mesh = plsc.VectorSubcoreMesh(core_axis_name="c", subcore_axis_name="s")

    @functools.partial(
        pl.kernel,
        mesh=mesh,
        out_type=jax.ShapeDtypeStruct((2, 16, STRIPE, d), jnp.float32),
        scratch_types=[
            pltpu.VMEM((NCHUNK, CHUNK), jnp.int32),
            pltpu.VMEM((NCHUNK, CHUNK), jnp.int32),
            pltpu.VMEM((CHUNK, d), jnp.float32),
            pltpu.VMEM((CHUNK, d), jnp.float32),
            pltpu.VMEM_SHARED((NPAD, d), jnp.float32),
            pltpu.SemaphoreType.DMA,
            pltpu.SemaphoreType.DMA,
        ],
    )
    def agg_kernel(g_hbm, src_hbm, dst_hbm, out_hbm, src_v, dst_v, ra, rb, acc, sa, sb):
        c = lax.axis_index("c")
        s = lax.axis_index("s")
        wid = c * 16 + s
        pltpu.sync_copy(src_hbm.at[wid], src_v)
        pltpu.sync_copy(dst_hbm.at[wid], dst_v)
        _fill_rows(ra, CHUNK, d, 0.0)
        _zero_stripe(acc, ra, s * STRIPE)
        plsc.subcore_barrier()
        bufs = (ra, rb)
        sems = (sa, sb)
        h = pltpu.async_copy(g_hbm.at[src_v.at[0]], ra, sa)
        for j in range(NCHUNK):
            cur = bufs[j % 2]
            h.wait()
            if j + 1 < NCHUNK:
                h = pltpu.async_copy(
                    g_hbm.at[src_v.at[j + 1]], bufs[(j + 1) % 2], sems[(j + 1) % 2]
                )
            pltpu.sync_copy(cur, acc.at[dst_v.at[j]], add=True)
        plsc.subcore_barrier()
        pltpu.sync_copy(acc.at[pl.ds(s * STRIPE, STRIPE)], out_hbm.at[c, s])

    return agg_kernel


_deg_kernel = _make_deg()
_agg_hid = _make_agg(HID)


def _tc_layer1(x_ref, w_ref, dp_ref, g_ref, dinv_ref):
    deg = dp_ref[0, :, 0:1] + dp_ref[1, :, 0:1] + 1.0
    dinv = lax.rsqrt(deg)
    h = jnp.dot(x_ref[...], w_ref[...], preferred_element_type=jnp.float32)
    g_ref[...] = h * dinv
    dinv_ref[...] = dinv


def _tc_layer2(agg_ref, g_ref, dinv_ref, b1_ref, w2_ref, g2_ref):
    dinv = dinv_ref[...]
    o = dinv * (agg_ref[0] + agg_ref[1] + g_ref[...]) + b1_ref[...]
    hrelu = jnp.maximum(o, 0.0)
    h2 = jnp.dot(hrelu, w2_ref[...], preferred_element_type=jnp.float32)
    g2_ref[...] = h2 * dinv


def _tc_decoder(agg_ref, g2_ref, dinv_ref, b2_ref, out_ref, rn_ref):
    i = pl.program_id(0)
    j = pl.program_id(1)

    @pl.when((i == 0) & (j == 0))
    def _():
        z = dinv_ref[...] * (agg_ref[0] + agg_ref[1] + g2_ref[...]) + b2_ref[...]
        z = z[:, 0:EMB]
        nrm = jnp.sqrt(jnp.sum(z * z, axis=1, keepdims=True))
        rn_ref[pl.ds(0, NPAD), :] = z / nrm
        rn_ref[pl.ds(NPAD, NPAD2 - NPAD), :] = jnp.zeros(
            (NPAD2 - NPAD, EMB), jnp.float32)

    a = rn_ref[pl.ds(i * BM, BM), :]
    b = rn_ref[pl.ds(j * BN, BN), :]
    sim = lax.dot_general(a, b, (((1,), (1,)), ((), ())),
                          preferred_element_type=jnp.float32)
    out_ref[...] = jax.nn.sigmoid(sim)


def kernel(x, edge_index, W1, b1, W2, b2):
    ei = edge_index.astype(jnp.int32)
    pad = jnp.full((EPAD - E,), N, jnp.int32)
    src = jnp.concatenate([ei[0], pad]).reshape(NTILES, NCHUNK, CHUNK)
    dst = jnp.concatenate([ei[1], pad]).reshape(NTILES, NCHUNK, CHUNK)
    x_pad = jnp.pad(x, ((0, NPAD - N), (0, 0)))

    degp = _deg_kernel(dst).reshape(2, NPAD, DEGW)

    g1, dinv = pl.pallas_call(
        _tc_layer1,
        out_shape=[
            jax.ShapeDtypeStruct((NPAD, HID), jnp.float32),
            jax.ShapeDtypeStruct((NPAD, 1), jnp.float32),
        ],
    )(x_pad, W1, degp)

    agg1 = _agg_hid(g1, src, dst).reshape(2, NPAD, HID)

    w2p = jnp.pad(W2, ((0, 0), (0, HID - EMB)))
    b2p = jnp.pad(b2, (0, HID - EMB)).reshape(1, HID)
    g2 = pl.pallas_call(
        _tc_layer2,
        out_shape=jax.ShapeDtypeStruct((NPAD, HID), jnp.float32),
    )(agg1, g1, dinv, b1.reshape(1, HID), w2p)

    agg2 = _agg_hid(g2, src, dst).reshape(2, NPAD, HID)

    out = pl.pallas_call(
        _tc_decoder,
        grid=(N // BM, pl.cdiv(N, BN)),
        in_specs=[
            pl.BlockSpec((2, NPAD, HID), lambda i, j: (0, 0, 0)),
            pl.BlockSpec((NPAD, HID), lambda i, j: (0, 0)),
            pl.BlockSpec((NPAD, 1), lambda i, j: (0, 0)),
            pl.BlockSpec((1, HID), lambda i, j: (0, 0)),
        ],
        out_specs=pl.BlockSpec((BM, BN), lambda i, j: (i, j)),
        out_shape=jax.ShapeDtypeStruct((N, N), jnp.float32),
        scratch_shapes=[pltpu.VMEM((NPAD2, EMB), jnp.float32)],
    )(agg2, g2, dinv, b2p)
    return out


# speedup vs baseline: 7.3162x; 1.0002x over previous
import functools

import jax
import jax.numpy as jnp
from jax import lax
from jax.experimental import pallas as pl
from jax.experimental.pallas import tpu as pltpu
from jax.experimental.pallas import tpu_sc as plsc

N = 10000
NPAD = 10016
STRIPE = NPAD // 16
E = 160000
NTILES = 32
NCHUNK = 40
CHUNK = 128
EPAD = NTILES * NCHUNK * CHUNK
IN_DIM = 128
HID = 128
EMB = 64
BM = 1000
BN = 1280
NPAD2 = 10240
DEGW = 16


def _fill_rows(buf, nrows, d, value):
    vec = jnp.full((16,), value, jnp.float32)

    def body(r, carry):
        for k in range(d // 16):
            buf[r, pl.ds(k * 16, 16)] = vec
        return carry

    lax.fori_loop(0, nrows, body, 0)


def _zero_stripe(acc, zbuf, base):
    for off in (0, 128, 256, 384):
        pltpu.sync_copy(zbuf, acc.at[pl.ds(base + off, CHUNK)])
    rem = STRIPE - 512
    pltpu.sync_copy(zbuf.at[pl.ds(0, rem)], acc.at[pl.ds(base + 512, rem)])


def _make_deg():
    mesh = plsc.VectorSubcoreMesh(core_axis_name="c", subcore_axis_name="s")

    @functools.partial(
        pl.kernel,
        mesh=mesh,
        out_type=jax.ShapeDtypeStruct((2, 16, STRIPE, DEGW), jnp.float32),
        scratch_types=[
            pltpu.VMEM((NCHUNK, CHUNK), jnp.int32),
            pltpu.VMEM((CHUNK, DEGW), jnp.float32),
            pltpu.VMEM((CHUNK, DEGW), jnp.float32),
            pltpu.VMEM_SHARED((NPAD, DEGW), jnp.float32),
        ],
    )
    def deg_kernel(dst_hbm, out_hbm, dst_v, ones_v, zero_v, acc):
        c = lax.axis_index("c")
        s = lax.axis_index("s")
        wid = c * 16 + s
        pltpu.sync_copy(dst_hbm.at[wid], dst_v)
        _fill_rows(ones_v, CHUNK, DEGW, 1.0)
        _fill_rows(zero_v, CHUNK, DEGW, 0.0)
        _zero_stripe(acc, zero_v, s * STRIPE)
        plsc.subcore_barrier()
        for j in range(NCHUNK):
            pltpu.sync_copy(ones_v, acc.at[dst_v.at[j]], add=True)
        plsc.subcore_barrier()
        pltpu.sync_copy(acc.at[pl.ds(s * STRIPE, STRIPE)], out_hbm.at[c, s])

    return deg_kernel


def _make_agg(d):
    mesh = plsc.VectorSubcoreMesh(core_axis_name="c", subcore_axis_name="s")

    @functools.partial(
        pl.kernel,
        mesh=mesh,
        out_type=jax.ShapeDtypeStruct((2, 16, STRIPE, d), jnp.float32),
        scratch_types=[
            pltpu.VMEM((NCHUNK, CHUNK), jnp.int32),
            pltpu.VMEM((NCHUNK, CHUNK), jnp.int32),
            pltpu.VMEM((CHUNK, d), jnp.float32),
            pltpu.VMEM((CHUNK, d), jnp.float32),
            pltpu.VMEM_SHARED((NPAD, d), jnp.float32),
            pltpu.SemaphoreType.DMA,
            pltpu.SemaphoreType.DMA,
            pltpu.SemaphoreType.DMA,
            pltpu.SemaphoreType.DMA,
        ],
    )
    def agg_kernel(g_hbm, src_hbm, dst_hbm, out_hbm, src_v, dst_v, ra, rb, acc,
                   sa, sb, sca, scb):
        c = lax.axis_index("c")
        s = lax.axis_index("s")
        wid = c * 16 + s
        pltpu.sync_copy(src_hbm.at[wid], src_v)
        pltpu.sync_copy(dst_hbm.at[wid], dst_v)
        _fill_rows(ra, CHUNK, d, 0.0)
        _zero_stripe(acc, ra, s * STRIPE)
        plsc.subcore_barrier()
        bufs = (ra, rb)
        sems = (sa, sb)
        scsems = (sca, scb)
        h = pltpu.async_copy(g_hbm.at[src_v.at[0]], ra, sa)
        sc_prev = None
        for j in range(NCHUNK):
            cur = bufs[j % 2]
            h.wait()
            if sc_prev is not None:
                sc_prev.wait()
            if j + 1 < NCHUNK:
                h = pltpu.async_copy(
                    g_hbm.at[src_v.at[j + 1]], bufs[(j + 1) % 2], sems[(j + 1) % 2]
                )
            sc_prev = pltpu.async_copy(cur, acc.at[dst_v.at[j]], scsems[j % 2],
                                       add=True)
        sc_prev.wait()
        plsc.subcore_barrier()
        pltpu.sync_copy(acc.at[pl.ds(s * STRIPE, STRIPE)], out_hbm.at[c, s])

    return agg_kernel


def _make_agg_sp(d):
    mesh = plsc.VectorSubcoreMesh(core_axis_name="c", subcore_axis_name="s")

    @functools.partial(
        pl.kernel,
        mesh=mesh,
        out_type=jax.ShapeDtypeStruct((2, 16, STRIPE, d), jnp.float32),
        scratch_types=[
            pltpu.VMEM((NCHUNK, CHUNK), jnp.int32),
            pltpu.VMEM((NCHUNK, CHUNK), jnp.int32),
            pltpu.VMEM((CHUNK, d), jnp.float32),
            pltpu.VMEM((CHUNK, d), jnp.float32),
            pltpu.VMEM_SHARED((NPAD, d), jnp.float32),
            pltpu.VMEM_SHARED((NPAD, d), jnp.float32),
            pltpu.SemaphoreType.DMA,
            pltpu.SemaphoreType.DMA,
            pltpu.SemaphoreType.DMA,
            pltpu.SemaphoreType.DMA,
        ],
    )
    def agg_kernel(g_hbm, src_hbm, dst_hbm, out_hbm, src_v, dst_v, ra, rb,
                   table, acc, sa, sb, sca, scb):
        c = lax.axis_index("c")
        s = lax.axis_index("s")
        wid = c * 16 + s
        pltpu.sync_copy(src_hbm.at[wid], src_v)
        pltpu.sync_copy(dst_hbm.at[wid], dst_v)
        pltpu.sync_copy(g_hbm.at[pl.ds(s * 624, 624)],
                        table.at[pl.ds(s * 624, 624)])

        @pl.when(s == 15)
        def _():
            pltpu.sync_copy(g_hbm.at[pl.ds(9984, NPAD - 9984)],
                            table.at[pl.ds(9984, NPAD - 9984)])
        _fill_rows(ra, CHUNK, d, 0.0)
        _zero_stripe(acc, ra, s * STRIPE)
        plsc.subcore_barrier()
        bufs = (ra, rb)
        sems = (sa, sb)
        scsems = (sca, scb)
        h = pltpu.async_copy(table.at[src_v.at[0]], ra, sa)
        sc_prev = None
        for j in range(NCHUNK):
            cur = bufs[j % 2]
            h.wait()
            if sc_prev is not None:
                sc_prev.wait()
            if j + 1 < NCHUNK:
                h = pltpu.async_copy(
                    table.at[src_v.at[j + 1]], bufs[(j + 1) % 2], sems[(j + 1) % 2]
                )
            sc_prev = pltpu.async_copy(cur, acc.at[dst_v.at[j]], scsems[j % 2],
                                       add=True)
        sc_prev.wait()
        plsc.subcore_barrier()
        pltpu.sync_copy(acc.at[pl.ds(s * STRIPE, STRIPE)], out_hbm.at[c, s])

    return agg_kernel


_deg_kernel = _make_deg()
_agg_hid = _make_agg(HID)
_agg_sp64 = _make_agg_sp(EMB)


def _tc_layer1(x_ref, w_ref, dp_ref, g_ref, dinv_ref):
    deg = dp_ref[0, :, 0:1] + dp_ref[1, :, 0:1] + 1.0
    dinv = lax.rsqrt(deg)
    h = jnp.dot(x_ref[...], w_ref[...], preferred_element_type=jnp.float32)
    g_ref[...] = h * dinv
    dinv_ref[...] = dinv


def _tc_layer2(agg_ref, g_ref, dinv_ref, b1_ref, w2_ref, g2_ref):
    dinv = dinv_ref[...]
    o = dinv * (agg_ref[0] + agg_ref[1] + g_ref[...]) + b1_ref[...]
    hrelu = jnp.maximum(o, 0.0)
    h2 = jnp.dot(hrelu, w2_ref[...], preferred_element_type=jnp.float32)
    g2_ref[...] = h2 * dinv


def _tc_decoder(agg_ref, g2_ref, dinv_ref, b2_ref, out_ref, rn_ref):
    i = pl.program_id(0)
    j = pl.program_id(1)

    @pl.when((i == 0) & (j == 0))
    def _():
        z = dinv_ref[...] * (agg_ref[0] + agg_ref[1] + g2_ref[...]) + b2_ref[...]
        z = z[:, 0:EMB]
        nrm = jnp.sqrt(jnp.sum(z * z, axis=1, keepdims=True))
        rn_ref[pl.ds(0, NPAD), :] = z / nrm
        rn_ref[pl.ds(NPAD, NPAD2 - NPAD), :] = jnp.zeros(
            (NPAD2 - NPAD, EMB), jnp.float32)

    a = rn_ref[pl.ds(i * BM, BM), :]
    b = rn_ref[pl.ds(j * BN, BN), :]
    sim = lax.dot_general(a, b, (((1,), (1,)), ((), ())),
                          preferred_element_type=jnp.float32)
    out_ref[...] = jax.nn.sigmoid(sim)


def kernel(x, edge_index, W1, b1, W2, b2):
    ei = edge_index.astype(jnp.int32)
    pad = jnp.full((EPAD - E,), N, jnp.int32)
    src = jnp.concatenate([ei[0], pad]).reshape(NTILES, NCHUNK, CHUNK)
    dst = jnp.concatenate([ei[1], pad]).reshape(NTILES, NCHUNK, CHUNK)
    x_pad = jnp.pad(x, ((0, NPAD - N), (0, 0)))

    degp = _deg_kernel(dst).reshape(2, NPAD, DEGW)

    g1, dinv = pl.pallas_call(
        _tc_layer1,
        out_shape=[
            jax.ShapeDtypeStruct((NPAD, HID), jnp.float32),
            jax.ShapeDtypeStruct((NPAD, 1), jnp.float32),
        ],
    )(x_pad, W1, degp)

    agg1 = _agg_hid(g1, src, dst).reshape(2, NPAD, HID)

    w2p = jnp.pad(W2, ((0, 0), (0, HID - EMB)))
    b2p = jnp.pad(b2, (0, HID - EMB)).reshape(1, HID)
    g2 = pl.pallas_call(
        _tc_layer2,
        out_shape=jax.ShapeDtypeStruct((NPAD, HID), jnp.float32),
    )(agg1, g1, dinv, b1.reshape(1, HID), w2p)

    agg2 = _agg_hid(g2, src, dst).reshape(2, NPAD, HID)

    out = pl.pallas_call(
        _tc_decoder,
        grid=(N // BM, pl.cdiv(N, BN)),
        in_specs=[
            pl.BlockSpec((2, NPAD, HID), lambda i, j: (0, 0, 0)),
            pl.BlockSpec((NPAD, HID), lambda i, j: (0, 0)),
            pl.BlockSpec((NPAD, 1), lambda i, j: (0, 0)),
            pl.BlockSpec((1, HID), lambda i, j: (0, 0)),
        ],
        out_specs=pl.BlockSpec((BM, BN), lambda i, j: (i, j)),
        out_shape=jax.ShapeDtypeStruct((N, N), jnp.float32),
        scratch_shapes=[pltpu.VMEM((NPAD2, EMB), jnp.float32)],
    )(agg2, g2, dinv, b2p)
    return out


# speedup vs baseline: 7.5413x; 1.0308x over previous
import functools

import jax
import jax.numpy as jnp
from jax import lax
from jax.experimental import pallas as pl
from jax.experimental.pallas import tpu as pltpu
from jax.experimental.pallas import tpu_sc as plsc

N = 10000
NPAD = 10016
STRIPE = NPAD // 16
E = 160000
NTILES = 32
NCHUNK = 40
CHUNK = 128
EPAD = NTILES * NCHUNK * CHUNK
IN_DIM = 128
HID = 128
EMB = 64
BM = 1000
BN = 1280
NPAD2 = 10240
DEGW = 16


def _fill_rows(buf, nrows, d, value):
    vec = jnp.full((16,), value, jnp.float32)

    def body(r, carry):
        for k in range(d // 16):
            buf[r, pl.ds(k * 16, 16)] = vec
        return carry

    lax.fori_loop(0, nrows, body, 0)


def _zero_stripe(acc, zbuf, base):
    nfull = STRIPE // CHUNK
    for k in range(nfull):
        pltpu.sync_copy(zbuf, acc.at[pl.ds(base + k * CHUNK, CHUNK)])
    rem = STRIPE - nfull * CHUNK
    if rem:
        pltpu.sync_copy(zbuf.at[pl.ds(0, rem)],
                        acc.at[pl.ds(base + nfull * CHUNK, rem)])


def _make_deg():
    mesh = plsc.VectorSubcoreMesh(core_axis_name="c", subcore_axis_name="s")

    @functools.partial(
        pl.kernel,
        mesh=mesh,
        out_type=jax.ShapeDtypeStruct((2, 16, STRIPE, DEGW), jnp.float32),
        scratch_types=[
            pltpu.VMEM((NCHUNK, CHUNK), jnp.int32),
            pltpu.VMEM((CHUNK, DEGW), jnp.float32),
            pltpu.VMEM((CHUNK, DEGW), jnp.float32),
            pltpu.VMEM_SHARED((NPAD, DEGW), jnp.float32),
        ],
    )
    def deg_kernel(dst_hbm, out_hbm, dst_v, ones_v, zero_v, acc):
        c = lax.axis_index("c")
        s = lax.axis_index("s")
        wid = c * 16 + s
        pltpu.sync_copy(dst_hbm.at[wid], dst_v)
        _fill_rows(ones_v, CHUNK, DEGW, 1.0)
        _fill_rows(zero_v, CHUNK, DEGW, 0.0)
        _zero_stripe(acc, zero_v, s * STRIPE)
        plsc.subcore_barrier()
        for j in range(NCHUNK):
            pltpu.sync_copy(ones_v, acc.at[dst_v.at[j]], add=True)
        plsc.subcore_barrier()
        pltpu.sync_copy(acc.at[pl.ds(s * STRIPE, STRIPE)], out_hbm.at[c, s])

    return deg_kernel


NBUF = 2


def _make_agg(d):
    mesh = plsc.VectorSubcoreMesh(core_axis_name="c", subcore_axis_name="s")

    @functools.partial(
        pl.kernel,
        mesh=mesh,
        out_type=jax.ShapeDtypeStruct((2, 16, STRIPE, d), jnp.float32),
        scratch_types=[
            pltpu.VMEM((NCHUNK, CHUNK), jnp.int32),
            pltpu.VMEM((NCHUNK, CHUNK), jnp.int32),
        ]
        + [pltpu.VMEM((CHUNK, d), jnp.float32)] * NBUF
        + [pltpu.VMEM_SHARED((NPAD, d), jnp.float32)]
        + [pltpu.SemaphoreType.DMA] * (2 * NBUF),
    )
    def agg_kernel(g_hbm, src_hbm, dst_hbm, out_hbm, src_v, dst_v, *rest):
        bufs = rest[:NBUF]
        acc = rest[NBUF]
        gsems = rest[NBUF + 1:]
        c = lax.axis_index("c")
        s = lax.axis_index("s")
        wid = c * 16 + s
        pltpu.sync_copy(src_hbm.at[wid], src_v)
        pltpu.sync_copy(dst_hbm.at[wid], dst_v)
        _fill_rows(bufs[0], CHUNK, d, 0.0)
        _zero_stripe(acc, bufs[0], s * STRIPE)
        plsc.subcore_barrier()

        HC = CHUNK // 2

        def gather_chunk(j):
            b = bufs[j % NBUF]
            return (
                pltpu.async_copy(g_hbm.at[src_v.at[j, pl.ds(0, HC)]],
                                 b.at[pl.ds(0, HC)], gsems[2 * (j % NBUF)]),
                pltpu.async_copy(g_hbm.at[src_v.at[j, pl.ds(HC, HC)]],
                                 b.at[pl.ds(HC, HC)], gsems[2 * (j % NBUF) + 1]),
            )

        gh = {0: gather_chunk(0), 1: gather_chunk(1)}
        for j in range(NCHUNK):
            gh[j][0].wait()
            gh[j][1].wait()
            pltpu.sync_copy(bufs[j % NBUF], acc.at[dst_v.at[j]], add=True)
            if j + 2 < NCHUNK:
                gh[j + 2] = gather_chunk(j + 2)
        plsc.subcore_barrier()
        pltpu.sync_copy(acc.at[pl.ds(s * STRIPE, STRIPE)], out_hbm.at[c, s])

    return agg_kernel


def _make_agg_sp(d):
    mesh = plsc.VectorSubcoreMesh(core_axis_name="c", subcore_axis_name="s")

    @functools.partial(
        pl.kernel,
        mesh=mesh,
        out_type=jax.ShapeDtypeStruct((2, 16, STRIPE, d), jnp.float32),
        scratch_types=[
            pltpu.VMEM((NCHUNK, CHUNK), jnp.int32),
            pltpu.VMEM((NCHUNK, CHUNK), jnp.int32),
            pltpu.VMEM((CHUNK, d), jnp.float32),
            pltpu.VMEM((CHUNK, d), jnp.float32),
            pltpu.VMEM_SHARED((NPAD, d), jnp.float32),
            pltpu.VMEM_SHARED((NPAD, d), jnp.float32),
            pltpu.SemaphoreType.DMA,
            pltpu.SemaphoreType.DMA,
            pltpu.SemaphoreType.DMA,
            pltpu.SemaphoreType.DMA,
        ],
    )
    def agg_kernel(g_hbm, src_hbm, dst_hbm, out_hbm, src_v, dst_v, ra, rb,
                   table, acc, sa, sb, sca, scb):
        c = lax.axis_index("c")
        s = lax.axis_index("s")
        wid = c * 16 + s
        pltpu.sync_copy(src_hbm.at[wid], src_v)
        pltpu.sync_copy(dst_hbm.at[wid], dst_v)
        pltpu.sync_copy(g_hbm.at[pl.ds(s * 624, 624)],
                        table.at[pl.ds(s * 624, 624)])

        @pl.when(s == 15)
        def _():
            pltpu.sync_copy(g_hbm.at[pl.ds(9984, NPAD - 9984)],
                            table.at[pl.ds(9984, NPAD - 9984)])
        _fill_rows(ra, CHUNK, d, 0.0)
        _zero_stripe(acc, ra, s * STRIPE)
        plsc.subcore_barrier()
        bufs = (ra, rb)
        sems = (sa, sb)
        scsems = (sca, scb)
        h = pltpu.async_copy(table.at[src_v.at[0]], ra, sa)
        sc_prev = None
        for j in range(NCHUNK):
            cur = bufs[j % 2]
            h.wait()
            if sc_prev is not None:
                sc_prev.wait()
            if j + 1 < NCHUNK:
                h = pltpu.async_copy(
                    table.at[src_v.at[j + 1]], bufs[(j + 1) % 2], sems[(j + 1) % 2]
                )
            sc_prev = pltpu.async_copy(cur, acc.at[dst_v.at[j]], scsems[j % 2],
                                       add=True)
        sc_prev.wait()
        plsc.subcore_barrier()
        pltpu.sync_copy(acc.at[pl.ds(s * STRIPE, STRIPE)], out_hbm.at[c, s])

    return agg_kernel


_deg_kernel = _make_deg()
_agg_hid = _make_agg(HID)
_agg_sp64 = _make_agg_sp(EMB)


def _tc_layer1(x_ref, w_ref, dp_ref, g_ref, dinv_ref):
    deg = dp_ref[0, :, 0:1] + dp_ref[1, :, 0:1] + 1.0
    dinv = lax.rsqrt(deg)
    h = jnp.dot(x_ref[...], w_ref[...], preferred_element_type=jnp.float32)
    g_ref[...] = h * dinv
    dinv_ref[...] = dinv


def _tc_layer2(agg_ref, g_ref, dinv_ref, b1_ref, w2_ref, g2_ref):
    dinv = dinv_ref[...]
    o = dinv * (agg_ref[0] + agg_ref[1] + g_ref[...]) + b1_ref[...]
    hrelu = jnp.maximum(o, 0.0)
    h2 = jnp.dot(hrelu, w2_ref[...], preferred_element_type=jnp.float32)
    g2_ref[...] = h2 * dinv


def _tc_decoder(agg_ref, g2_ref, dinv_ref, b2_ref, out_ref, rn_ref):
    i = pl.program_id(0)
    j = pl.program_id(1)

    @pl.when((i == 0) & (j == 0))
    def _():
        z = dinv_ref[...] * (agg_ref[0] + agg_ref[1] + g2_ref[...]) + b2_ref[...]
        z = z[:, 0:EMB]
        nrm = jnp.sqrt(jnp.sum(z * z, axis=1, keepdims=True))
        rn_ref[pl.ds(0, NPAD), :] = z / nrm
        rn_ref[pl.ds(NPAD, NPAD2 - NPAD), :] = jnp.zeros(
            (NPAD2 - NPAD, EMB), jnp.float32)

    a = rn_ref[pl.ds(i * BM, BM), :]
    b = rn_ref[pl.ds(j * BN, BN), :]
    sim = lax.dot_general(a, b, (((1,), (1,)), ((), ())),
                          preferred_element_type=jnp.float32)
    out_ref[...] = jax.nn.sigmoid(sim)


def kernel(x, edge_index, W1, b1, W2, b2):
    ei = edge_index.astype(jnp.int32)
    pad = jnp.full((EPAD - E,), N, jnp.int32)
    src = jnp.concatenate([ei[0], pad]).reshape(NTILES, NCHUNK, CHUNK)
    dst = jnp.concatenate([ei[1], pad]).reshape(NTILES, NCHUNK, CHUNK)
    x_pad = jnp.pad(x, ((0, NPAD - N), (0, 0)))

    degp = _deg_kernel(dst).reshape(2, NPAD, DEGW)

    g1, dinv = pl.pallas_call(
        _tc_layer1,
        out_shape=[
            jax.ShapeDtypeStruct((NPAD, HID), jnp.float32),
            jax.ShapeDtypeStruct((NPAD, 1), jnp.float32),
        ],
    )(x_pad, W1, degp)

    agg1 = _agg_hid(g1, src, dst).reshape(2, NPAD, HID)

    w2p = jnp.pad(W2, ((0, 0), (0, HID - EMB)))
    b2p = jnp.pad(b2, (0, HID - EMB)).reshape(1, HID)
    g2 = pl.pallas_call(
        _tc_layer2,
        out_shape=jax.ShapeDtypeStruct((NPAD, HID), jnp.float32),
    )(agg1, g1, dinv, b1.reshape(1, HID), w2p)

    agg2 = _agg_hid(g2, src, dst).reshape(2, NPAD, HID)

    out = pl.pallas_call(
        _tc_decoder,
        grid=(N // BM, pl.cdiv(N, BN)),
        in_specs=[
            pl.BlockSpec((2, NPAD, HID), lambda i, j: (0, 0, 0)),
            pl.BlockSpec((NPAD, HID), lambda i, j: (0, 0)),
            pl.BlockSpec((NPAD, 1), lambda i, j: (0, 0)),
            pl.BlockSpec((1, HID), lambda i, j: (0, 0)),
        ],
        out_specs=pl.BlockSpec((BM, BN), lambda i, j: (i, j)),
        out_shape=jax.ShapeDtypeStruct((N, N), jnp.float32),
        scratch_shapes=[pltpu.VMEM((NPAD2, EMB), jnp.float32)],
    )(agg2, g2, dinv, b2p)
    return out


# speedup vs baseline: 7.9077x; 1.0486x over previous
import functools

import jax
import jax.numpy as jnp
from jax import lax
from jax.experimental import pallas as pl
from jax.experimental.pallas import tpu as pltpu
from jax.experimental.pallas import tpu_sc as plsc

N = 10000
NPAD = 10016
STRIPE = NPAD // 16
E = 160000
NTILES = 32
NCHUNK = 40
CHUNK = 128
EPAD = NTILES * NCHUNK * CHUNK
IN_DIM = 128
HID = 128
EMB = 64
BM = 1000
BN = 1280
NPAD2 = 10240
DEGW = 16


def _fill_rows(buf, nrows, d, value):
    vec = jnp.full((16,), value, jnp.float32)

    def body(r, carry):
        for k in range(d // 16):
            buf[r, pl.ds(k * 16, 16)] = vec
        return carry

    lax.fori_loop(0, nrows, body, 0)


def _zero_stripe(acc, zbuf, base):
    nfull = STRIPE // CHUNK
    for k in range(nfull):
        pltpu.sync_copy(zbuf, acc.at[pl.ds(base + k * CHUNK, CHUNK)])
    rem = STRIPE - nfull * CHUNK
    if rem:
        pltpu.sync_copy(zbuf.at[pl.ds(0, rem)],
                        acc.at[pl.ds(base + nfull * CHUNK, rem)])


def _make_deg():
    mesh = plsc.VectorSubcoreMesh(core_axis_name="c", subcore_axis_name="s")

    @functools.partial(
        pl.kernel,
        mesh=mesh,
        out_type=jax.ShapeDtypeStruct((2, 16, STRIPE, DEGW), jnp.float32),
        scratch_types=[
            pltpu.VMEM((NCHUNK, CHUNK), jnp.int32),
            pltpu.VMEM((CHUNK, DEGW), jnp.float32),
            pltpu.VMEM((CHUNK, DEGW), jnp.float32),
            pltpu.VMEM_SHARED((NPAD, DEGW), jnp.float32),
        ],
    )
    def deg_kernel(dst_hbm, out_hbm, dst_v, ones_v, zero_v, acc):
        c = lax.axis_index("c")
        s = lax.axis_index("s")
        wid = c * 16 + s
        pltpu.sync_copy(dst_hbm.at[pl.ds(wid * NCHUNK, NCHUNK)], dst_v)
        _fill_rows(ones_v, CHUNK, DEGW, 1.0)
        _fill_rows(zero_v, CHUNK, DEGW, 0.0)
        _zero_stripe(acc, zero_v, s * STRIPE)
        plsc.subcore_barrier()
        for j in range(NCHUNK):
            pltpu.sync_copy(ones_v, acc.at[dst_v.at[j]], add=True)
        plsc.subcore_barrier()
        pltpu.sync_copy(acc.at[pl.ds(s * STRIPE, STRIPE)], out_hbm.at[c, s])

    return deg_kernel


NBUF = 2

NC0 = 64
NC1 = 16


def _make_agg(d):
    mesh = plsc.VectorSubcoreMesh(core_axis_name="c", subcore_axis_name="s")

    @functools.partial(
        pl.kernel,
        mesh=mesh,
        out_type=jax.ShapeDtypeStruct((2, 16, STRIPE, d), jnp.float32),
        scratch_types=[
            pltpu.VMEM((NC0, CHUNK), jnp.int32),
            pltpu.VMEM((NC0, CHUNK), jnp.int32),
        ]
        + [pltpu.VMEM((CHUNK, d), jnp.float32)] * NBUF
        + [pltpu.VMEM_SHARED((NPAD, d), jnp.float32)]
        + [pltpu.SemaphoreType.DMA] * (2 * NBUF),
    )
    def agg_kernel(g_hbm, src_hbm, dst_hbm, out_hbm, src_v, dst_v, *rest):
        bufs = rest[:NBUF]
        acc = rest[NBUF]
        gsems = rest[NBUF + 1:]
        c = lax.axis_index("c")
        s = lax.axis_index("s")
        _fill_rows(bufs[0], CHUNK, d, 0.0)
        _zero_stripe(acc, bufs[0], s * STRIPE)

        HC = CHUNK // 2

        def run_chunks(start, count):
            pltpu.sync_copy(src_hbm.at[pl.ds(start, count)],
                            src_v.at[pl.ds(0, count)])
            pltpu.sync_copy(dst_hbm.at[pl.ds(start, count)],
                            dst_v.at[pl.ds(0, count)])

            def gather_chunk(j):
                b = bufs[j % NBUF]
                return (
                    pltpu.async_copy(g_hbm.at[src_v.at[j, pl.ds(0, HC)]],
                                     b.at[pl.ds(0, HC)], gsems[2 * (j % NBUF)]),
                    pltpu.async_copy(g_hbm.at[src_v.at[j, pl.ds(HC, HC)]],
                                     b.at[pl.ds(HC, HC)],
                                     gsems[2 * (j % NBUF) + 1]),
                )

            gh = {0: gather_chunk(0), 1: gather_chunk(1)}
            for j in range(count):
                gh[j][0].wait()
                gh[j][1].wait()
                pltpu.sync_copy(bufs[j % NBUF], acc.at[dst_v.at[j]], add=True)
                if j + 2 < count:
                    gh[j + 2] = gather_chunk(j + 2)

        plsc.subcore_barrier()

        @pl.when(c == 0)
        def _():
            run_chunks(s * NC0, NC0)

        @pl.when(c == 1)
        def _():
            run_chunks(16 * NC0 + s * NC1, NC1)

        plsc.subcore_barrier()
        pltpu.sync_copy(acc.at[pl.ds(s * STRIPE, STRIPE)], out_hbm.at[c, s])

    return agg_kernel


def _make_agg_sp(d):
    mesh = plsc.VectorSubcoreMesh(core_axis_name="c", subcore_axis_name="s")

    @functools.partial(
        pl.kernel,
        mesh=mesh,
        out_type=jax.ShapeDtypeStruct((2, 16, STRIPE, d), jnp.float32),
        scratch_types=[
            pltpu.VMEM((NCHUNK, CHUNK), jnp.int32),
            pltpu.VMEM((NCHUNK, CHUNK), jnp.int32),
            pltpu.VMEM((CHUNK, d), jnp.float32),
            pltpu.VMEM((CHUNK, d), jnp.float32),
            pltpu.VMEM_SHARED((NPAD, d), jnp.float32),
            pltpu.VMEM_SHARED((NPAD, d), jnp.float32),
            pltpu.SemaphoreType.DMA,
            pltpu.SemaphoreType.DMA,
            pltpu.SemaphoreType.DMA,
            pltpu.SemaphoreType.DMA,
        ],
    )
    def agg_kernel(g_hbm, src_hbm, dst_hbm, out_hbm, src_v, dst_v, ra, rb,
                   table, acc, sa, sb, sca, scb):
        c = lax.axis_index("c")
        s = lax.axis_index("s")
        wid = c * 16 + s
        pltpu.sync_copy(src_hbm.at[wid], src_v)
        pltpu.sync_copy(dst_hbm.at[wid], dst_v)
        pltpu.sync_copy(g_hbm.at[pl.ds(s * 624, 624)],
                        table.at[pl.ds(s * 624, 624)])

        @pl.when(s == 15)
        def _():
            pltpu.sync_copy(g_hbm.at[pl.ds(9984, NPAD - 9984)],
                            table.at[pl.ds(9984, NPAD - 9984)])
        _fill_rows(ra, CHUNK, d, 0.0)
        _zero_stripe(acc, ra, s * STRIPE)
        plsc.subcore_barrier()
        bufs = (ra, rb)
        sems = (sa, sb)
        scsems = (sca, scb)
        h = pltpu.async_copy(table.at[src_v.at[0]], ra, sa)
        sc_prev = None
        for j in range(NCHUNK):
            cur = bufs[j % 2]
            h.wait()
            if sc_prev is not None:
                sc_prev.wait()
            if j + 1 < NCHUNK:
                h = pltpu.async_copy(
                    table.at[src_v.at[j + 1]], bufs[(j + 1) % 2], sems[(j + 1) % 2]
                )
            sc_prev = pltpu.async_copy(cur, acc.at[dst_v.at[j]], scsems[j % 2],
                                       add=True)
        sc_prev.wait()
        plsc.subcore_barrier()
        pltpu.sync_copy(acc.at[pl.ds(s * STRIPE, STRIPE)], out_hbm.at[c, s])

    return agg_kernel


_deg_kernel = _make_deg()
_agg_hid = _make_agg(HID)
_agg_sp64 = _make_agg_sp(EMB)


def _tc_layer1(x_ref, w_ref, dp_ref, g_ref, dinv_ref):
    deg = dp_ref[0, :, 0:1] + dp_ref[1, :, 0:1] + 1.0
    dinv = lax.rsqrt(deg)
    h = jnp.dot(x_ref[...], w_ref[...], preferred_element_type=jnp.float32)
    g_ref[...] = h * dinv
    dinv_ref[...] = dinv


def _tc_layer2(agg_ref, g_ref, dinv_ref, b1_ref, w2_ref, g2_ref):
    dinv = dinv_ref[...]
    o = dinv * (agg_ref[0] + agg_ref[1] + g_ref[...]) + b1_ref[...]
    hrelu = jnp.maximum(o, 0.0)
    h2 = jnp.dot(hrelu, w2_ref[...], preferred_element_type=jnp.float32)
    g2_ref[...] = h2 * dinv


def _tc_decoder(agg_ref, g2_ref, dinv_ref, b2_ref, out_ref, rn_ref):
    i = pl.program_id(0)
    j = pl.program_id(1)

    @pl.when((i == 0) & (j == 0))
    def _():
        z = dinv_ref[...] * (agg_ref[0] + agg_ref[1] + g2_ref[...]) + b2_ref[...]
        z = z[:, 0:EMB]
        nrm = jnp.sqrt(jnp.sum(z * z, axis=1, keepdims=True))
        rn_ref[pl.ds(0, NPAD), :] = z / nrm
        rn_ref[pl.ds(NPAD, NPAD2 - NPAD), :] = jnp.zeros(
            (NPAD2 - NPAD, EMB), jnp.float32)

    a = rn_ref[pl.ds(i * BM, BM), :]
    b = rn_ref[pl.ds(j * BN, BN), :]
    sim = lax.dot_general(a, b, (((1,), (1,)), ((), ())),
                          preferred_element_type=jnp.float32)
    out_ref[...] = jax.nn.sigmoid(sim)


def kernel(x, edge_index, W1, b1, W2, b2):
    ei = edge_index.astype(jnp.int32)
    pad = jnp.full((EPAD - E,), N, jnp.int32)
    src = jnp.concatenate([ei[0], pad]).reshape(NTILES * NCHUNK, CHUNK)
    dst = jnp.concatenate([ei[1], pad]).reshape(NTILES * NCHUNK, CHUNK)
    x_pad = jnp.pad(x, ((0, NPAD - N), (0, 0)))

    degp = _deg_kernel(dst).reshape(2, NPAD, DEGW)

    g1, dinv = pl.pallas_call(
        _tc_layer1,
        out_shape=[
            jax.ShapeDtypeStruct((NPAD, HID), jnp.float32),
            jax.ShapeDtypeStruct((NPAD, 1), jnp.float32),
        ],
    )(x_pad, W1, degp)

    agg1 = _agg_hid(g1, src, dst).reshape(2, NPAD, HID)

    w2p = jnp.pad(W2, ((0, 0), (0, HID - EMB)))
    b2p = jnp.pad(b2, (0, HID - EMB)).reshape(1, HID)
    g2 = pl.pallas_call(
        _tc_layer2,
        out_shape=jax.ShapeDtypeStruct((NPAD, HID), jnp.float32),
    )(agg1, g1, dinv, b1.reshape(1, HID), w2p)

    agg2 = _agg_hid(g2, src, dst).reshape(2, NPAD, HID)

    out = pl.pallas_call(
        _tc_decoder,
        grid=(N // BM, pl.cdiv(N, BN)),
        in_specs=[
            pl.BlockSpec((2, NPAD, HID), lambda i, j: (0, 0, 0)),
            pl.BlockSpec((NPAD, HID), lambda i, j: (0, 0)),
            pl.BlockSpec((NPAD, 1), lambda i, j: (0, 0)),
            pl.BlockSpec((1, HID), lambda i, j: (0, 0)),
        ],
        out_specs=pl.BlockSpec((BM, BN), lambda i, j: (i, j)),
        out_shape=jax.ShapeDtypeStruct((N, N), jnp.float32),
        scratch_shapes=[pltpu.VMEM((NPAD2, EMB), jnp.float32)],
    )(agg2, g2, dinv, b2p)
    return out
